# Initial kernel scaffold; baseline (speedup 1.0000x reference)
import jax
import jax.numpy as jnp
from jax.experimental import pallas as pl


def kernel(idx, G):
    raise NotImplementedError("write your pallas kernel here")


# speedup vs baseline: 8.2559x; 8.2559x over previous
import functools

import jax
import jax.numpy as jnp
from jax import lax
from jax.experimental import pallas as pl
from jax.experimental.pallas import tpu as pltpu
from jax.experimental.pallas import tpu_sc as plsc

D_MODEL = 128
VOCAB = 11
NC = 2
NS = 16
NW = NC * NS

CHUNK = 128


def _sc_body(nchunks, g_hbm, idx_hbm, out_hbm, table_sh, idx_v, rows_v, sem):
  cid = lax.axis_index("c")
  sid = lax.axis_index("s")
  wid = sid * NC + cid
  per_w = nchunks * CHUNK
  base = wid * per_w

  @pl.when(sid == 0)
  def _():
    pltpu.sync_copy(g_hbm, table_sh)

  plsc.subcore_barrier()

  def step(i, carry):
    start = base + i * CHUNK
    pltpu.sync_copy(idx_hbm.at[pl.ds(start, CHUNK)], idx_v)
    pltpu.async_copy(table_sh.at[idx_v], rows_v, sem).wait()
    pltpu.sync_copy(rows_v, out_hbm.at[pl.ds(start, CHUNK)])
    return carry

  lax.fori_loop(0, nchunks, step, 0)


@jax.jit
def kernel(idx, G):
  b0, b1 = idx.shape
  n = b0 * b1
  assert n % (NW * CHUNK) == 0
  nchunks = n // (NW * CHUNK)
  idx_flat = idx.reshape(n).astype(jnp.int32)

  mesh = plsc.VectorSubcoreMesh(core_axis_name="c", subcore_axis_name="s")
  out = pl.kernel(
      functools.partial(_sc_body, nchunks),
      out_type=jax.ShapeDtypeStruct((n, D_MODEL), jnp.float32),
      mesh=mesh,
      scratch_types=[
          pltpu.VMEM_SHARED((VOCAB, D_MODEL), jnp.float32),
          pltpu.VMEM((CHUNK,), jnp.int32),
          pltpu.VMEM((CHUNK, D_MODEL), jnp.float32),
          pltpu.SemaphoreType.DMA,
      ],
  )(G, idx_flat)
  return out.reshape(b0, b1, D_MODEL)


# speedup vs baseline: 17.9742x; 2.1771x over previous
import jax
import jax.numpy as jnp
from jax import lax
from jax.experimental import pallas as pl
from jax.experimental.pallas import tpu as pltpu
from jax.experimental.pallas import tpu_sc as plsc

D_MODEL = 128
VOCAB = 11
NC = 2
NS = 16
NW = NC * NS

CHUNK = 128
NCH_BLK = 200
IDXBLK = CHUNK * NCH_BLK
NBLK = 4
PER_W = IDXBLK * NBLK


def _sc_body(g_hbm, idx_hbm, out_hbm, table_sh, idxb0, idxb1, rows0, rows1,
             sem_i, sem_g, sem_o0, sem_o1):
  cid = lax.axis_index("c")
  sid = lax.axis_index("s")
  wid = sid * NC + cid
  base = wid * PER_W

  @pl.when(sid == 0)
  def _():
    pltpu.sync_copy(g_hbm, table_sh)

  plsc.subcore_barrier()

  idxbufs = (idxb0, idxb1)
  rows = (rows0, rows1)
  sem_o = (sem_o0, sem_o1)

  def chunk(idxbuf, blk_row0, c, b, wait_out):
    if wait_out:
      pltpu.make_async_copy(rows[b], out_hbm.at[pl.ds(0, CHUNK)],
                            sem_o[b]).wait()
    row0 = blk_row0 + c * CHUNK
    pltpu.async_copy(table_sh.at[idxbuf.at[pl.ds(c * CHUNK, CHUNK)]],
                     rows[b], sem_g).wait()
    pltpu.async_copy(rows[b], out_hbm.at[pl.ds(row0, CHUNK)], sem_o[b])

  pltpu.async_copy(idx_hbm.at[pl.ds(base, IDXBLK)], idxbufs[0], sem_i)

  for blk in range(NBLK):
    cur = blk % 2
    idxbuf = idxbufs[cur]
    blk_row0 = base + blk * IDXBLK
    pltpu.make_async_copy(idx_hbm.at[pl.ds(base, IDXBLK)], idxbuf,
                          sem_i).wait()
    if blk < NBLK - 1:
      pltpu.async_copy(idx_hbm.at[pl.ds(blk_row0 + IDXBLK, IDXBLK)],
                       idxbufs[1 - cur], sem_i)
    lo = 0
    if blk == 0:
      chunk(idxbuf, blk_row0, 0, 0, False)
      chunk(idxbuf, blk_row0, 1, 1, False)
      lo = 1

    @pl.loop(lo, NCH_BLK // 2)
    def _(k):
      for b in range(2):
        chunk(idxbuf, blk_row0, 2 * k + b, b, True)

  pltpu.make_async_copy(rows0, out_hbm.at[pl.ds(0, CHUNK)], sem_o0).wait()
  pltpu.make_async_copy(rows1, out_hbm.at[pl.ds(0, CHUNK)], sem_o1).wait()


@jax.jit
def kernel(idx, G):
  b0, b1 = idx.shape
  n = b0 * b1
  assert n == NW * PER_W
  idx_flat = idx.reshape(n).astype(jnp.int32)

  mesh = plsc.VectorSubcoreMesh(core_axis_name="c", subcore_axis_name="s")
  out = pl.kernel(
      _sc_body,
      out_type=jax.ShapeDtypeStruct((n, D_MODEL), jnp.float32),
      mesh=mesh,
      scratch_types=[
          pltpu.VMEM_SHARED((VOCAB, D_MODEL), jnp.float32),
          pltpu.VMEM((IDXBLK,), jnp.int32),
          pltpu.VMEM((IDXBLK,), jnp.int32),
          pltpu.VMEM((CHUNK, D_MODEL), jnp.float32),
          pltpu.VMEM((CHUNK, D_MODEL), jnp.float32),
          pltpu.SemaphoreType.DMA,
          pltpu.SemaphoreType.DMA,
          pltpu.SemaphoreType.DMA,
          pltpu.SemaphoreType.DMA,
      ],
  )(G, idx_flat)
  return out.reshape(b0, b1, D_MODEL)


# speedup vs baseline: 18.6505x; 1.0376x over previous
import jax
import jax.numpy as jnp
from jax import lax
from jax.experimental import pallas as pl
from jax.experimental.pallas import tpu as pltpu
from jax.experimental.pallas import tpu_sc as plsc

D_MODEL = 128
VOCAB = 11
NC = 2
NS = 16
NW = NC * NS

GCHUNK = 128
GPC = 2
CHUNK = GCHUNK * GPC
NCH_BLK = 100
IDXBLK = CHUNK * NCH_BLK
NBLK = 4
PER_W = IDXBLK * NBLK


def _sc_body(g_hbm, idx_hbm, out_hbm, table_sh, idxb0, idxb1, rows0, rows1,
             sem_i, sem_g, sem_o0, sem_o1):
  cid = lax.axis_index("c")
  sid = lax.axis_index("s")
  wid = sid * NC + cid
  base = wid * PER_W

  @pl.when(sid == 0)
  def _():
    pltpu.sync_copy(g_hbm, table_sh)

  plsc.subcore_barrier()

  idxbufs = (idxb0, idxb1)
  rows = (rows0, rows1)
  sem_o = (sem_o0, sem_o1)

  def chunk(idxbuf, blk_row0, c, b, wait_out):
    if wait_out:
      pltpu.make_async_copy(rows[b], out_hbm.at[pl.ds(0, CHUNK)],
                            sem_o[b]).wait()
    row0 = blk_row0 + c * CHUNK
    descs = []
    for g in range(GPC):
      descs.append(pltpu.async_copy(
          table_sh.at[idxbuf.at[pl.ds(c * CHUNK + g * GCHUNK, GCHUNK)]],
          rows[b].at[pl.ds(g * GCHUNK, GCHUNK)], sem_g))
    for d in descs:
      d.wait()
    pltpu.async_copy(rows[b], out_hbm.at[pl.ds(row0, CHUNK)], sem_o[b])

  pltpu.async_copy(idx_hbm.at[pl.ds(base, IDXBLK)], idxbufs[0], sem_i)

  for blk in range(NBLK):
    cur = blk % 2
    idxbuf = idxbufs[cur]
    blk_row0 = base + blk * IDXBLK
    pltpu.make_async_copy(idx_hbm.at[pl.ds(base, IDXBLK)], idxbuf,
                          sem_i).wait()
    if blk < NBLK - 1:
      pltpu.async_copy(idx_hbm.at[pl.ds(blk_row0 + IDXBLK, IDXBLK)],
                       idxbufs[1 - cur], sem_i)
    lo = 0
    if blk == 0:
      chunk(idxbuf, blk_row0, 0, 0, False)
      chunk(idxbuf, blk_row0, 1, 1, False)
      lo = 1

    @pl.loop(lo, NCH_BLK // 2)
    def _(k):
      for b in range(2):
        chunk(idxbuf, blk_row0, 2 * k + b, b, True)

  pltpu.make_async_copy(rows0, out_hbm.at[pl.ds(0, CHUNK)], sem_o0).wait()
  pltpu.make_async_copy(rows1, out_hbm.at[pl.ds(0, CHUNK)], sem_o1).wait()


@jax.jit
def kernel(idx, G):
  b0, b1 = idx.shape
  n = b0 * b1
  assert n == NW * PER_W
  idx_flat = idx.reshape(n).astype(jnp.int32)

  mesh = plsc.VectorSubcoreMesh(core_axis_name="c", subcore_axis_name="s")
  out = pl.kernel(
      _sc_body,
      out_type=jax.ShapeDtypeStruct((n, D_MODEL), jnp.float32),
      mesh=mesh,
      scratch_types=[
          pltpu.VMEM_SHARED((VOCAB, D_MODEL), jnp.float32),
          pltpu.VMEM((IDXBLK,), jnp.int32),
          pltpu.VMEM((IDXBLK,), jnp.int32),
          pltpu.VMEM((CHUNK, D_MODEL), jnp.float32),
          pltpu.VMEM((CHUNK, D_MODEL), jnp.float32),

          pltpu.SemaphoreType.DMA,
          pltpu.SemaphoreType.DMA,
          pltpu.SemaphoreType.DMA,
          pltpu.SemaphoreType.DMA,
      ],
  )(G, idx_flat)
  return out.reshape(b0, b1, D_MODEL)


# speedup vs baseline: 18.7027x; 1.0028x over previous
import jax
import jax.numpy as jnp
from jax import lax
from jax.experimental import pallas as pl
from jax.experimental.pallas import tpu as pltpu
from jax.experimental.pallas import tpu_sc as plsc

D_MODEL = 128
VOCAB = 11
NC = 2
NS = 16
NW = NC * NS

GCHUNK = 128
GPC = 2
CHUNK = GCHUNK * GPC
NCH_BLK = 100
IDXBLK = CHUNK * NCH_BLK
NBLK = 4
PER_W = IDXBLK * NBLK


def _sc_body(g_hbm, idx_hbm, out_hbm, table_sh, idxb0, idxb1, rows0, rows1,
             sem_i, sem_g, sem_o0, sem_o1):
  cid = lax.axis_index("c")
  sid = lax.axis_index("s")
  wid = sid * NC + cid
  base = wid * PER_W

  pltpu.sync_copy(g_hbm, table_sh.at[pl.ds(sid * VOCAB, VOCAB)])
  plsc.subcore_barrier()
  off = sid * VOCAB

  idxbufs = (idxb0, idxb1)
  rows = (rows0, rows1)
  sem_o = (sem_o0, sem_o1)

  def chunk(idxbuf, blk_row0, c, b, wait_out):
    if wait_out:
      pltpu.make_async_copy(rows[b], out_hbm.at[pl.ds(0, CHUNK)],
                            sem_o[b]).wait()
    row0 = blk_row0 + c * CHUNK
    descs = []
    for g in range(GPC):
      descs.append(pltpu.async_copy(
          table_sh.at[idxbuf.at[pl.ds(c * CHUNK + g * GCHUNK, GCHUNK)]],
          rows[b].at[pl.ds(g * GCHUNK, GCHUNK)], sem_g))
    for d in descs:
      d.wait()
    pltpu.async_copy(rows[b], out_hbm.at[pl.ds(row0, CHUNK)], sem_o[b])

  pltpu.async_copy(idx_hbm.at[pl.ds(base, IDXBLK)], idxbufs[0], sem_i)

  for blk in range(NBLK):
    cur = blk % 2
    idxbuf = idxbufs[cur]
    blk_row0 = base + blk * IDXBLK
    pltpu.make_async_copy(idx_hbm.at[pl.ds(base, IDXBLK)], idxbuf,
                          sem_i).wait()

    @pl.loop(0, IDXBLK // 16, unroll=8)
    def _(j):
      idxbuf[pl.ds(j * 16, 16)] = idxbuf[pl.ds(j * 16, 16)] + off

    if blk < NBLK - 1:
      pltpu.async_copy(idx_hbm.at[pl.ds(blk_row0 + IDXBLK, IDXBLK)],
                       idxbufs[1 - cur], sem_i)
    lo = 0
    if blk == 0:
      chunk(idxbuf, blk_row0, 0, 0, False)
      chunk(idxbuf, blk_row0, 1, 1, False)
      lo = 1

    @pl.loop(lo, NCH_BLK // 2)
    def _(k):
      for b in range(2):
        chunk(idxbuf, blk_row0, 2 * k + b, b, True)

  pltpu.make_async_copy(rows0, out_hbm.at[pl.ds(0, CHUNK)], sem_o0).wait()
  pltpu.make_async_copy(rows1, out_hbm.at[pl.ds(0, CHUNK)], sem_o1).wait()


@jax.jit
def kernel(idx, G):
  b0, b1 = idx.shape
  n = b0 * b1
  assert n == NW * PER_W
  idx_flat = idx.reshape(n).astype(jnp.int32)

  mesh = plsc.VectorSubcoreMesh(core_axis_name="c", subcore_axis_name="s")
  out = pl.kernel(
      _sc_body,
      out_type=jax.ShapeDtypeStruct((n, D_MODEL), jnp.float32),
      mesh=mesh,
      scratch_types=[
          pltpu.VMEM_SHARED((NS * VOCAB, D_MODEL), jnp.float32),
          pltpu.VMEM((IDXBLK,), jnp.int32),
          pltpu.VMEM((IDXBLK,), jnp.int32),
          pltpu.VMEM((CHUNK, D_MODEL), jnp.float32),
          pltpu.VMEM((CHUNK, D_MODEL), jnp.float32),

          pltpu.SemaphoreType.DMA,
          pltpu.SemaphoreType.DMA,
          pltpu.SemaphoreType.DMA,
          pltpu.SemaphoreType.DMA,
      ],
  )(G, idx_flat)
  return out.reshape(b0, b1, D_MODEL)


# speedup vs baseline: 19.6228x; 1.0492x over previous
import jax
import jax.numpy as jnp
from jax import lax
from jax.experimental import pallas as pl
from jax.experimental.pallas import tpu as pltpu
from jax.experimental.pallas import tpu_sc as plsc

D_MODEL = 128
VOCAB = 11
NC = 2
NS = 16
NW = NC * NS

CHUNK = 128
NBUF = 4
NCH_BLK = 200
IDXBLK = CHUNK * NCH_BLK
NBLK = 4
PER_W = IDXBLK * NBLK


def _sc_body(g_hbm, idx_hbm, out_hbm, table_sh, idxb0, idxb1,
             rows0, rows1, rows2, rows3,
             sem_i, sem_g, sem_o0, sem_o1, sem_o2, sem_o3):
  cid = lax.axis_index("c")
  sid = lax.axis_index("s")
  wid = sid * NC + cid
  base = wid * PER_W

  pltpu.sync_copy(g_hbm, table_sh.at[pl.ds(sid * VOCAB, VOCAB)])
  plsc.subcore_barrier()
  off = sid * VOCAB

  idxbufs = (idxb0, idxb1)
  rows = (rows0, rows1, rows2, rows3)
  sem_o = (sem_o0, sem_o1, sem_o2, sem_o3)

  def wait_out(s):
    pltpu.make_async_copy(rows[s], out_hbm.at[pl.ds(0, CHUNK)],
                          sem_o[s]).wait()

  def g_start(idxbuf, c, s):
    pltpu.async_copy(table_sh.at[idxbuf.at[pl.ds(c * CHUNK, CHUNK)]],
                     rows[s], sem_g)

  def g_wait(s):
    pltpu.make_async_copy(table_sh.at[idxbufs[0].at[pl.ds(0, CHUNK)]],
                          rows[s], sem_g).wait()

  def o_start(blk_row0, c, s):
    pltpu.async_copy(rows[s], out_hbm.at[pl.ds(blk_row0 + c * CHUNK, CHUNK)],
                     sem_o[s])

  def body(idxbuf, blk_row0, c, b, do_wait_out=True, lookahead=True):
    sw = (b + 1) % NBUF
    if lookahead:
      if do_wait_out:
        wait_out(sw)
      g_start(idxbuf, c + 1, sw)
    g_wait(b)
    o_start(blk_row0, c, b)

  pltpu.async_copy(idx_hbm.at[pl.ds(base, IDXBLK)], idxbufs[0], sem_i)

  for blk in range(NBLK):
    cur = blk % 2
    idxbuf = idxbufs[cur]
    blk_row0 = base + blk * IDXBLK
    pltpu.make_async_copy(idx_hbm.at[pl.ds(base, IDXBLK)], idxbuf,
                          sem_i).wait()

    @pl.loop(0, IDXBLK // 16, unroll=8)
    def _(j):
      idxbuf[pl.ds(j * 16, 16)] = idxbuf[pl.ds(j * 16, 16)] + off

    if blk < NBLK - 1:
      pltpu.async_copy(idx_hbm.at[pl.ds(blk_row0 + IDXBLK, IDXBLK)],
                       idxbufs[1 - cur], sem_i)

    if blk > 0:
      wait_out(0)
    g_start(idxbuf, 0, 0)

    lo = 1
    if blk == 0:
      body(idxbuf, blk_row0, 0, 0, do_wait_out=False)
      body(idxbuf, blk_row0, 1, 1, do_wait_out=False)
      body(idxbuf, blk_row0, 2, 2, do_wait_out=False)
      body(idxbuf, blk_row0, 3, 3)
    else:
      lo = 0

    @pl.loop(lo, NCH_BLK // NBUF - 1)
    def _(k):
      for b in range(NBUF):
        body(idxbuf, blk_row0, NBUF * k + b, b)

    c0 = NCH_BLK - NBUF
    body(idxbuf, blk_row0, c0 + 0, 0)
    body(idxbuf, blk_row0, c0 + 1, 1)
    body(idxbuf, blk_row0, c0 + 2, 2)
    body(idxbuf, blk_row0, c0 + 3, 3, lookahead=False)

  for s in range(NBUF):
    wait_out(s)


@jax.jit
def kernel(idx, G):
  b0, b1 = idx.shape
  n = b0 * b1
  assert n == NW * PER_W
  idx_flat = idx.reshape(n).astype(jnp.int32)

  mesh = plsc.VectorSubcoreMesh(core_axis_name="c", subcore_axis_name="s")
  out = pl.kernel(
      _sc_body,
      out_type=jax.ShapeDtypeStruct((n, D_MODEL), jnp.float32),
      mesh=mesh,
      scratch_types=[
          pltpu.VMEM_SHARED((NS * VOCAB, D_MODEL), jnp.float32),
          pltpu.VMEM((IDXBLK,), jnp.int32),
          pltpu.VMEM((IDXBLK,), jnp.int32),
          pltpu.VMEM((CHUNK, D_MODEL), jnp.float32),
          pltpu.VMEM((CHUNK, D_MODEL), jnp.float32),
          pltpu.VMEM((CHUNK, D_MODEL), jnp.float32),
          pltpu.VMEM((CHUNK, D_MODEL), jnp.float32),
          pltpu.SemaphoreType.DMA,
          pltpu.SemaphoreType.DMA,
          pltpu.SemaphoreType.DMA,
          pltpu.SemaphoreType.DMA,
          pltpu.SemaphoreType.DMA,
          pltpu.SemaphoreType.DMA,
      ],
  )(G, idx_flat)
  return out.reshape(b0, b1, D_MODEL)


# speedup vs baseline: 19.9170x; 1.0150x over previous
import jax
import jax.numpy as jnp
from jax import lax
from jax.experimental import pallas as pl
from jax.experimental.pallas import tpu as pltpu
from jax.experimental.pallas import tpu_sc as plsc

D_MODEL = 128
VOCAB = 11
NC = 2
NS = 16
NW = NC * NS

ROW = 200
G1 = 128
G2 = ROW - G1
NBUF = 4
RBLK = 32
NROWS = 16384
ROWS_W = NROWS // NW
NBLK = ROWS_W // RBLK


def _sc_body(g_hbm, idx_hbm, out_hbm, table_sh, idxb0, idxb1,
             rows0, rows1, rows2, rows3,
             sem_i, sem_g, sem_o0, sem_o1, sem_o2, sem_o3):
  cid = lax.axis_index("c")
  sid = lax.axis_index("s")
  wid = sid * NC + cid
  base_row = wid * ROWS_W

  @pl.when(sid == 0)
  def _():
    pltpu.sync_copy(g_hbm, table_sh)

  plsc.subcore_barrier()

  idxbufs = (idxb0, idxb1)
  rows = (rows0, rows1, rows2, rows3)
  sem_o = (sem_o0, sem_o1, sem_o2, sem_o3)

  def wait_out(s):
    pltpu.make_async_copy(rows[s], out_hbm.at[pl.ds(0, ROW)],
                          sem_o[s]).wait()

  def g_start(idxbuf, r, s):
    pltpu.async_copy(table_sh.at[idxbuf.at[r, pl.ds(0, G1)]],
                     rows[s].at[pl.ds(0, G1)], sem_g)
    pltpu.async_copy(table_sh.at[idxbuf.at[r, pl.ds(G1, G2)]],
                     rows[s].at[pl.ds(G1, G2)], sem_g)

  def g_wait(s):
    pltpu.make_async_copy(table_sh.at[idxbufs[0].at[0, pl.ds(0, G1)]],
                          rows[s].at[pl.ds(0, G1)], sem_g).wait()
    pltpu.make_async_copy(table_sh.at[idxbufs[0].at[0, pl.ds(G1, G2)]],
                          rows[s].at[pl.ds(G1, G2)], sem_g).wait()

  def o_start(blk_row0, r, s):
    pltpu.async_copy(rows[s],
                     out_hbm.at[pl.ds((blk_row0 + r) * ROW, ROW)], sem_o[s])

  def body(idxbuf, blk_row0, r, b, do_wait_out=True, lookahead=True):
    sw = (b + 1) % NBUF
    if lookahead:
      if do_wait_out:
        wait_out(sw)
      g_start(idxbuf, r + 1, sw)
    g_wait(b)
    o_start(blk_row0, r, b)

  pltpu.async_copy(idx_hbm.at[pl.ds(base_row, RBLK)], idxbufs[0], sem_i)

  for blk in range(NBLK):
    cur = blk % 2
    idxbuf = idxbufs[cur]
    blk_row0 = base_row + blk * RBLK
    pltpu.make_async_copy(idx_hbm.at[pl.ds(base_row, RBLK)], idxbuf,
                          sem_i).wait()
    if blk < NBLK - 1:
      pltpu.async_copy(idx_hbm.at[pl.ds(blk_row0 + RBLK, RBLK)],
                       idxbufs[1 - cur], sem_i)

    if blk > 0:
      wait_out(0)
    g_start(idxbuf, 0, 0)

    lo = 1
    if blk == 0:
      body(idxbuf, blk_row0, 0, 0, do_wait_out=False)
      body(idxbuf, blk_row0, 1, 1, do_wait_out=False)
      body(idxbuf, blk_row0, 2, 2, do_wait_out=False)
      body(idxbuf, blk_row0, 3, 3)
    else:
      lo = 0

    @pl.loop(lo, RBLK // NBUF - 1)
    def _(k):
      for b in range(NBUF):
        body(idxbuf, blk_row0, NBUF * k + b, b)

    r0 = RBLK - NBUF
    body(idxbuf, blk_row0, r0 + 0, 0)
    body(idxbuf, blk_row0, r0 + 1, 1)
    body(idxbuf, blk_row0, r0 + 2, 2)
    body(idxbuf, blk_row0, r0 + 3, 3, lookahead=False)

  for s in range(NBUF):
    wait_out(s)


@jax.jit
def kernel(idx, G):
  b0, b1 = idx.shape
  assert b0 == NROWS and b1 == ROW
  n = b0 * b1

  mesh = plsc.VectorSubcoreMesh(core_axis_name="c", subcore_axis_name="s")
  out = pl.kernel(
      _sc_body,
      out_type=jax.ShapeDtypeStruct((n, D_MODEL), jnp.float32),
      mesh=mesh,
      scratch_types=[
          pltpu.VMEM_SHARED((VOCAB, D_MODEL), jnp.float32),
          pltpu.VMEM((RBLK, ROW), jnp.int32),
          pltpu.VMEM((RBLK, ROW), jnp.int32),
          pltpu.VMEM((ROW, D_MODEL), jnp.float32),
          pltpu.VMEM((ROW, D_MODEL), jnp.float32),
          pltpu.VMEM((ROW, D_MODEL), jnp.float32),
          pltpu.VMEM((ROW, D_MODEL), jnp.float32),
          pltpu.SemaphoreType.DMA,
          pltpu.SemaphoreType.DMA,
          pltpu.SemaphoreType.DMA,
          pltpu.SemaphoreType.DMA,
          pltpu.SemaphoreType.DMA,
          pltpu.SemaphoreType.DMA,
      ],
  )(G, idx.astype(jnp.int32))
  return out.reshape(b0, b1, D_MODEL)
